# baseline (speedup 1.0000x reference)
import functools

import jax
import jax.numpy as jnp
from jax.experimental import pallas as pl
from jax.experimental.pallas import tpu as pltpu

_B, _T, _D = 16, 1024, 32
_K = 8192
_TT = 256
_BT = _B * _T
_G = _BT // _TT

_PREC = jax.lax.Precision.HIGHEST


def _body(x_ref, cb_ref, p0_ref, p1_ref, recon_ref, loss_ref):
    x = x_ref[...]
    cb0 = cb_ref[0]
    cb1 = cb_ref[1]

    def soft_assign(res, cb):
        n = jnp.sum(cb * cb, axis=1)[None, :]
        s = 2.0 * jax.lax.dot_general(
            res, cb, (((1,), (1,)), ((), ())),
            preferred_element_type=jnp.float32, precision=_PREC) - n
        m = jnp.max(s, axis=1, keepdims=True)
        e = jnp.exp(s - m)
        d = jnp.sum(e, axis=1, keepdims=True)
        p = e / d
        q = jax.lax.dot_general(
            p, cb, (((1,), (0,)), ((), ())),
            preferred_element_type=jnp.float32, precision=_PREC)
        return p, q

    p0, q0 = soft_assign(x, cb0)
    r1 = x - q0
    p1, q1 = soft_assign(r1, cb1)
    recon = q0 + q1

    p0_ref[...] = p0
    p1_ref[...] = p1
    recon_ref[...] = recon
    dr = recon - x
    loss_ref[...] = (jnp.sum(r1 * r1) + jnp.sum(dr * dr)).reshape(1, 1, 1)


@jax.jit
def kernel(x, codebooks):
    x2 = x.reshape(_BT, _D)
    p0, p1, recon2, losses = pl.pallas_call(
        _body,
        grid=(_G,),
        in_specs=[
            pl.BlockSpec((_TT, _D), lambda i: (i, 0)),
            pl.BlockSpec((2, _K, _D), lambda i: (0, 0, 0)),
        ],
        out_specs=[
            pl.BlockSpec((_TT, _K), lambda i: (i, 0)),
            pl.BlockSpec((_TT, _K), lambda i: (i, 0)),
            pl.BlockSpec((_TT, _D), lambda i: (i, 0)),
            pl.BlockSpec((1, 1, 1), lambda i: (i, 0, 0)),
        ],
        out_shape=[
            jax.ShapeDtypeStruct((_BT, _K), jnp.float32),
            jax.ShapeDtypeStruct((_BT, _K), jnp.float32),
            jax.ShapeDtypeStruct((_BT, _D), jnp.float32),
            jax.ShapeDtypeStruct((_G, 1, 1), jnp.float32),
        ],
        compiler_params=pltpu.CompilerParams(
            dimension_semantics=("parallel",),
        ),
    )(x2, codebooks)
    index_probs = jnp.stack(
        [p0.reshape(_B, _T, _K), p1.reshape(_B, _T, _K)], axis=-1)
    recon = recon2.reshape(_B, _T, _D)
    loss = jnp.sum(losses) * (1.25 / (_BT * _D))
    return recon, index_probs, loss


# speedup vs baseline: 2.0574x; 2.0574x over previous
import functools

import jax
import jax.numpy as jnp
from jax.experimental import pallas as pl
from jax.experimental.pallas import tpu as pltpu

_B, _T, _D = 16, 1024, 32
_K = 8192
_TT = 256
_BT = _B * _T
_G = _BT // _TT

_PREC = jax.lax.Precision.DEFAULT


def _body(x_ref, cb_ref, p0_ref, p1_ref, recon_ref, loss_ref):
    x = x_ref[...]
    cb0 = cb_ref[0]
    cb1 = cb_ref[1]

    def soft_assign(res, cb):
        n = jnp.sum(cb * cb, axis=1)[None, :]
        s = 2.0 * jax.lax.dot_general(
            res, cb, (((1,), (1,)), ((), ())),
            preferred_element_type=jnp.float32, precision=_PREC) - n
        m = jnp.max(s, axis=1, keepdims=True)
        e = jnp.exp(s - m)
        d = jnp.sum(e, axis=1, keepdims=True)
        p = e / d
        q = jax.lax.dot_general(
            p, cb, (((1,), (0,)), ((), ())),
            preferred_element_type=jnp.float32, precision=_PREC)
        return p, q

    p0, q0 = soft_assign(x, cb0)
    r1 = x - q0
    p1, q1 = soft_assign(r1, cb1)
    recon = q0 + q1

    p0_ref[...] = p0
    p1_ref[...] = p1
    recon_ref[...] = recon
    dr = recon - x
    loss_ref[...] = (jnp.sum(r1 * r1) + jnp.sum(dr * dr)).reshape(1, 1, 1)


@jax.jit
def kernel(x, codebooks):
    x2 = x.reshape(_BT, _D)
    p0, p1, recon2, losses = pl.pallas_call(
        _body,
        grid=(_G,),
        in_specs=[
            pl.BlockSpec((_TT, _D), lambda i: (i, 0)),
            pl.BlockSpec((2, _K, _D), lambda i: (0, 0, 0)),
        ],
        out_specs=[
            pl.BlockSpec((_TT, _K), lambda i: (i, 0)),
            pl.BlockSpec((_TT, _K), lambda i: (i, 0)),
            pl.BlockSpec((_TT, _D), lambda i: (i, 0)),
            pl.BlockSpec((1, 1, 1), lambda i: (i, 0, 0)),
        ],
        out_shape=[
            jax.ShapeDtypeStruct((_BT, _K), jnp.float32),
            jax.ShapeDtypeStruct((_BT, _K), jnp.float32),
            jax.ShapeDtypeStruct((_BT, _D), jnp.float32),
            jax.ShapeDtypeStruct((_G, 1, 1), jnp.float32),
        ],
        compiler_params=pltpu.CompilerParams(
            dimension_semantics=("parallel",),
        ),
    )(x2, codebooks)
    index_probs = jnp.stack(
        [p0.reshape(_B, _T, _K), p1.reshape(_B, _T, _K)], axis=-1)
    recon = recon2.reshape(_B, _T, _D)
    loss = jnp.sum(losses) * (1.25 / (_BT * _D))
    return recon, index_probs, loss
